# Initial kernel scaffold; baseline (speedup 1.0000x reference)
import jax
import jax.numpy as jnp
from jax.experimental import pallas as pl


def kernel(x, W_edge, b_edge, W_fusion, W_offset, b_offset):
    raise NotImplementedError("write your pallas kernel here")


# speedup vs baseline: 6.8956x; 6.8956x over previous
import functools

import jax
import jax.numpy as jnp
from jax import lax
from jax.experimental import pallas as pl
from jax.experimental.pallas import tpu as pltpu

B, V, N, D, K = 2, 4, 196, 384, 8
NV = V - 1
M = V * N
NP = 256


def _scores_body(xq_ref, xk_ref, out_ref):
    xq = xq_ref[0, 0]
    xk = xk_ref[0, 0]
    gram = jax.lax.dot_general(xq, xk, (((1,), (1,)), ((), ())),
                               preferred_element_type=jnp.float32)
    ss = jnp.sum(xk * xk, axis=1)[None, :]
    s = ss - 2.0 * gram
    pad = jnp.full((N, NP - N), jnp.inf, jnp.float32)
    out_ref[0, 0, 0] = jnp.concatenate([s, pad], axis=1)


def _scores(x):
    grid = (B, V, NV)
    return pl.pallas_call(
        _scores_body,
        grid=grid,
        in_specs=[
            pl.BlockSpec((1, 1, N, D), lambda b, v, j: (b, v, 0, 0)),
            pl.BlockSpec((1, 1, N, D),
                         lambda b, v, j: (b, j + (j >= v).astype(j.dtype), 0, 0)),
        ],
        out_specs=pl.BlockSpec((1, 1, 1, N, NP),
                               lambda b, v, j: (b, v, j, 0, 0)),
        out_shape=jax.ShapeDtypeStruct((B, V, NV, N, NP), jnp.float32),
    )(x, x)


def _proj_body(x_ref, we_ref, wo1_ref, bo_ref, u_ref, p_ref):
    xb = x_ref[0, 0]
    u_ref[0, 0] = jax.lax.dot_general(xb, we_ref[...], (((1,), (0,)), ((), ())),
                                      preferred_element_type=jnp.float32)
    p_ref[0, 0] = xb + jax.lax.dot_general(
        xb, wo1_ref[...], (((1,), (0,)), ((), ())),
        preferred_element_type=jnp.float32) + bo_ref[...][None, :]


def _proj(x, W_edge, Wo1, b_offset):
    grid = (B, V)
    return pl.pallas_call(
        _proj_body,
        grid=grid,
        in_specs=[
            pl.BlockSpec((1, 1, N, D), lambda b, v: (b, v, 0, 0)),
            pl.BlockSpec((D, D), lambda b, v: (0, 0)),
            pl.BlockSpec((D, D), lambda b, v: (0, 0)),
            pl.BlockSpec((D,), lambda b, v: (0,)),
        ],
        out_specs=[
            pl.BlockSpec((1, 1, N, D), lambda b, v: (b, v, 0, 0)),
            pl.BlockSpec((1, 1, N, D), lambda b, v: (b, v, 0, 0)),
        ],
        out_shape=[
            jax.ShapeDtypeStruct((B, V, N, D), jnp.float32),
            jax.ShapeDtypeStruct((B, V, N, D), jnp.float32),
        ],
    )(x, W_edge, Wo1, b_offset)


def _topk_gather_jnp(scores, U):
    s = scores[..., :N]
    order = jnp.argsort(s, axis=-1)[..., :K]
    topk = jnp.sort(order, axis=-1)
    v_idx = jnp.arange(V)[None, :, None, None, None]
    j_idx = jnp.arange(NV)[None, None, :, None, None]
    v1 = j_idx + (j_idx >= v_idx)
    gidx = v1 * N + topk
    gidx = jnp.moveaxis(gidx, 3, 2)
    gidx = gidx.reshape(B, M, NV * K)
    Uf = U.reshape(B, M, D)
    Ug = Uf[jnp.arange(B)[:, None, None], gidx]
    return Ug


QC = 49


def _fusion_body(ug_ref, u_ref, p_ref, wf_ref, wo2_ref, be_ref, out_ref):
    ug = ug_ref[0]
    uq = u_ref[0]
    pre = (ug.reshape(QC, NV * K, D) - uq[:, None, :] + be_ref[...][None, None, :])
    edge = 0.5 * pre * (1.0 + lax.erf(pre * (2.0 ** -0.5)))
    logits = jax.lax.dot_general(
        edge.reshape(QC * NV * K, D), wf_ref[...], (((1,), (0,)), ((), ())),
        preferred_element_type=jnp.float32).reshape(QC, NV * K, D)
    mx = jnp.max(logits, axis=1, keepdims=True)
    e = jnp.exp(logits - mx)
    w = e / jnp.sum(e, axis=1, keepdims=True)
    edge_sum = jnp.sum(edge * w, axis=1)
    out_ref[0] = p_ref[0] + jax.lax.dot_general(
        edge_sum, wo2_ref[...], (((1,), (0,)), ((), ())),
        preferred_element_type=jnp.float32)


def _fusion(Ug, U, P, W_fusion, Wo2, b_edge):
    R = B * M
    grid = (R // QC,)
    return pl.pallas_call(
        _fusion_body,
        grid=grid,
        in_specs=[
            pl.BlockSpec((1, QC * NV * K, D), lambda i: (i, 0, 0)),
            pl.BlockSpec((1, QC, D), lambda i: (i, 0, 0)),
            pl.BlockSpec((1, QC, D), lambda i: (i, 0, 0)),
            pl.BlockSpec((D, D), lambda i: (0, 0)),
            pl.BlockSpec((D, D), lambda i: (0, 0)),
            pl.BlockSpec((D,), lambda i: (0,)),
        ],
        out_specs=pl.BlockSpec((1, QC, D), lambda i: (i, 0, 0)),
        out_shape=jax.ShapeDtypeStruct((R // QC, QC, D), jnp.float32),
    )(Ug.reshape(R // QC, QC * NV * K, D),
      U.reshape(R // QC, QC, D),
      P.reshape(R // QC, QC, D),
      W_fusion, Wo2, b_edge)


def kernel(x, W_edge, b_edge, W_fusion, W_offset, b_offset):
    scores = _scores(x)
    U, P = _proj(x, W_edge, W_offset[:D], b_offset)
    Ug = _topk_gather_jnp(scores, U)
    out = _fusion(Ug.reshape(B * M, NV * K, D),
                  U.reshape(B * M, D),
                  P.reshape(B * M, D),
                  W_fusion, W_offset[D:], b_edge)
    return out.reshape(B, V, N, D)


# speedup vs baseline: 13.9687x; 2.0257x over previous
import functools

import jax
import jax.numpy as jnp
from jax import lax
from jax.experimental import pallas as pl
from jax.experimental.pallas import tpu as pltpu
from jax.experimental.pallas import tpu_sc as plsc

B, V, N, D, K = 2, 4, 196, 384, 8
NV = V - 1
M = V * N
NP = 256


def _scores_body(xq_ref, xk_ref, out_ref):
    xq = xq_ref[0, 0]
    xk = xk_ref[0, 0]
    gram = jax.lax.dot_general(xq, xk, (((1,), (1,)), ((), ())),
                               preferred_element_type=jnp.float32)
    ss = jnp.sum(xk * xk, axis=1)[None, :]
    s = ss - 2.0 * gram
    pad = jnp.full((N, NP - N), jnp.inf, jnp.float32)
    out_ref[0, 0, 0] = jnp.concatenate([s, pad], axis=1)


def _scores(x):
    grid = (B, V, NV)
    return pl.pallas_call(
        _scores_body,
        grid=grid,
        in_specs=[
            pl.BlockSpec((1, 1, N, D), lambda b, v, j: (b, v, 0, 0)),
            pl.BlockSpec((1, 1, N, D),
                         lambda b, v, j: (b, j + (j >= v).astype(j.dtype), 0, 0)),
        ],
        out_specs=pl.BlockSpec((1, 1, 1, N, NP),
                               lambda b, v, j: (b, v, j, 0, 0)),
        out_shape=jax.ShapeDtypeStruct((B, V, NV, N, NP), jnp.float32),
    )(x, x)


def _proj_body(x_ref, we_ref, wo1_ref, bo_ref, u_ref, p_ref):
    xb = x_ref[0, 0]
    u_ref[0, 0] = jax.lax.dot_general(xb, we_ref[...], (((1,), (0,)), ((), ())),
                                      preferred_element_type=jnp.float32)
    p_ref[0, 0] = xb + jax.lax.dot_general(
        xb, wo1_ref[...], (((1,), (0,)), ((), ())),
        preferred_element_type=jnp.float32) + bo_ref[...][None, :]


def _proj(x, W_edge, Wo1, b_offset):
    grid = (B, V)
    return pl.pallas_call(
        _proj_body,
        grid=grid,
        in_specs=[
            pl.BlockSpec((1, 1, N, D), lambda b, v: (b, v, 0, 0)),
            pl.BlockSpec((D, D), lambda b, v: (0, 0)),
            pl.BlockSpec((D, D), lambda b, v: (0, 0)),
            pl.BlockSpec((D,), lambda b, v: (0,)),
        ],
        out_specs=[
            pl.BlockSpec((1, 1, N, D), lambda b, v: (b, v, 0, 0)),
            pl.BlockSpec((1, 1, N, D), lambda b, v: (b, v, 0, 0)),
        ],
        out_shape=[
            jax.ShapeDtypeStruct((B, V, N, D), jnp.float32),
            jax.ShapeDtypeStruct((B, V, N, D), jnp.float32),
        ],
    )(x, W_edge, Wo1, b_offset)


QT = 49
NCH = NP // 16
GQ = 4


def _sc_topk_gather_body(scores_hbm, u_hbm, ug_hbm, b0, b1, b2, idx_v,
                         rows_v, rows_tail, sem):
    nc = 2
    wid = lax.axis_index("s") * nc + lax.axis_index("c")
    q0 = wid * QT
    b = q0 // M
    v = (q0 % M) // N
    n0 = q0 % N
    bufs = (b0, b1, b2)
    offs = []
    for j in range(NV):
        row0 = ((b * V + v) * NV + j) * N + n0
        al = (row0 // 8) * 8
        offs.append(row0 - al)
        pltpu.sync_copy(scores_hbm.at[pl.ds(al, QT + 7)], bufs[j])

    lane = lax.iota(jnp.int32, 16)
    inf16 = jnp.full((16,), jnp.inf, jnp.float32)
    zero16 = jnp.zeros((16,), jnp.int32)
    sentinel = jnp.where(lane < K, 0, jnp.int32(2**30))

    def per_query(qq, _):
        def per_chunk(c, st):
            out = []
            for j in range(NV):
                bv, bi = st[2 * j], st[2 * j + 1]
                vals = bufs[j][qq + offs[j], pl.ds(c * 16, 16)]
                idxs = c * 16 + lane
                sv, si = plsc.sort_key_val(vals, idxs)
                rv = lax.rev(sv, (0,))
                ri = lax.rev(si, (0,))
                keep = bv <= rv
                mv = jnp.where(keep, bv, rv)
                mi = jnp.where(keep, bi, ri)
                nbv, nbi = plsc.sort_key_val(mv, mi)
                out.extend([nbv, nbi])
            return tuple(out)

        st = (inf16, zero16) * NV
        st = lax.fori_loop(0, NCH, per_chunk, st)
        for j in range(NV):
            v1 = j + jnp.where(j >= v, 1, 0)
            base = b * M + v1 * N
            keys = st[2 * j + 1] + sentinel
            gidx, _ = plsc.sort_key_val(keys, keys)
            plsc.store_compressed(idx_v.at[pl.ds(qq * (NV * K) + j * K, 16)],
                                  gidx + base, mask=lane < K)
        return 0

    lax.fori_loop(0, QT, per_query, 0)

    out0 = q0 * NV * K
    for ch in range(QT // GQ):
        cp = pltpu.async_copy(
            u_hbm.at[idx_v.at[pl.ds(ch * GQ * NV * K, GQ * NV * K)]],
            rows_v, sem)
        cp.wait()
        pltpu.sync_copy(rows_v, ug_hbm.at[pl.ds(out0 + ch * GQ * NV * K,
                                                GQ * NV * K)])
    tail0 = (QT // GQ) * GQ * NV * K
    cp = pltpu.async_copy(u_hbm.at[idx_v.at[pl.ds(tail0, NV * K)]],
                          rows_tail, sem)
    cp.wait()
    pltpu.sync_copy(rows_tail, ug_hbm.at[pl.ds(out0 + tail0, NV * K)])


def _topk_gather_sc(scores, U):
    scores2 = scores.reshape(B * V * NV * N, NP)
    U2 = U.reshape(B * M, D)
    mesh = plsc.VectorSubcoreMesh(core_axis_name="c", subcore_axis_name="s")
    fn = pl.kernel(
        _sc_topk_gather_body,
        out_type=jax.ShapeDtypeStruct((B * M * NV * K, D), jnp.float32),
        mesh=mesh,
        scratch_types=[
            pltpu.VMEM((QT + 7, NP), jnp.float32),
            pltpu.VMEM((QT + 7, NP), jnp.float32),
            pltpu.VMEM((QT + 7, NP), jnp.float32),
            pltpu.VMEM((QT * NV * K + 8,), jnp.int32),
            pltpu.VMEM((GQ * NV * K, D), jnp.float32),
            pltpu.VMEM((NV * K, D), jnp.float32),
            pltpu.SemaphoreType.DMA,
        ],
        compiler_params=pltpu.CompilerParams(needs_layout_passes=False),
    )
    return fn(scores2, U2)


QC = 49


def _fusion_body(ug_ref, u_ref, p_ref, wf_ref, wo2_ref, be_ref, out_ref):
    ug = ug_ref[0]
    uq = u_ref[0]
    pre = (ug.reshape(QC, NV * K, D) - uq[:, None, :] + be_ref[...][None, None, :])
    edge = 0.5 * pre * (1.0 + lax.erf(pre * (2.0 ** -0.5)))
    logits = jax.lax.dot_general(
        edge.reshape(QC * NV * K, D), wf_ref[...], (((1,), (0,)), ((), ())),
        preferred_element_type=jnp.float32).reshape(QC, NV * K, D)
    mx = jnp.max(logits, axis=1, keepdims=True)
    e = jnp.exp(logits - mx)
    w = e / jnp.sum(e, axis=1, keepdims=True)
    edge_sum = jnp.sum(edge * w, axis=1)
    out_ref[0] = p_ref[0] + jax.lax.dot_general(
        edge_sum, wo2_ref[...], (((1,), (0,)), ((), ())),
        preferred_element_type=jnp.float32)


def _fusion(Ug, U, P, W_fusion, Wo2, b_edge):
    R = B * M
    grid = (R // QC,)
    return pl.pallas_call(
        _fusion_body,
        grid=grid,
        in_specs=[
            pl.BlockSpec((1, QC * NV * K, D), lambda i: (i, 0, 0)),
            pl.BlockSpec((1, QC, D), lambda i: (i, 0, 0)),
            pl.BlockSpec((1, QC, D), lambda i: (i, 0, 0)),
            pl.BlockSpec((D, D), lambda i: (0, 0)),
            pl.BlockSpec((D, D), lambda i: (0, 0)),
            pl.BlockSpec((D,), lambda i: (0,)),
        ],
        out_specs=pl.BlockSpec((1, QC, D), lambda i: (i, 0, 0)),
        out_shape=jax.ShapeDtypeStruct((R // QC, QC, D), jnp.float32),
    )(Ug.reshape(R // QC, QC * NV * K, D),
      U.reshape(R // QC, QC, D),
      P.reshape(R // QC, QC, D),
      W_fusion, Wo2, b_edge)


def kernel(x, W_edge, b_edge, W_fusion, W_offset, b_offset):
    scores = _scores(x)
    U, P = _proj(x, W_edge, W_offset[:D], b_offset)
    Ug = _topk_gather_sc(scores, U)
    out = _fusion(Ug.reshape(B * M, NV * K, D),
                  U.reshape(B * M, D),
                  P.reshape(B * M, D),
                  W_fusion, W_offset[D:], b_edge)
    return out.reshape(B, V, N, D)


# speedup vs baseline: 31.7680x; 2.2742x over previous
import functools

import jax
import jax.numpy as jnp
from jax import lax
from jax.experimental import pallas as pl
from jax.experimental.pallas import tpu as pltpu
from jax.experimental.pallas import tpu_sc as plsc

B, V, N, D, K = 2, 4, 196, 384, 8
NV = V - 1
M = V * N
NP = 256


def _scores_body(xq_ref, xk_ref, out_ref):
    xq = xq_ref[0, 0]
    xk = xk_ref[0, 0]
    gram = jax.lax.dot_general(xq, xk, (((1,), (1,)), ((), ())),
                               preferred_element_type=jnp.float32)
    ones_row = jnp.ones((8, D), jnp.float32)
    ss = jax.lax.dot_general(ones_row, xk * xk, (((1,), (1,)), ((), ())),
                             preferred_element_type=jnp.float32)[:1]
    s = ss - 2.0 * gram
    pad = jnp.full((N, NP - N), jnp.inf, jnp.float32)
    out_ref[0, 0, 0] = jnp.concatenate([s, pad], axis=1)


def _scores(x):
    grid = (B, V, NV)
    return pl.pallas_call(
        _scores_body,
        grid=grid,
        in_specs=[
            pl.BlockSpec((1, 1, N, D), lambda b, v, j: (b, v, 0, 0)),
            pl.BlockSpec((1, 1, N, D),
                         lambda b, v, j: (b, j + (j >= v).astype(j.dtype), 0, 0)),
        ],
        out_specs=pl.BlockSpec((1, 1, 1, N, NP),
                               lambda b, v, j: (b, v, j, 0, 0)),
        out_shape=jax.ShapeDtypeStruct((B, V, NV, N, NP), jnp.float32),
    )(x, x)


def _proj_body(x_ref, we_ref, wo1_ref, bo_ref, u_ref, p_ref):
    xb = x_ref[0, 0]
    u_ref[0, 0] = jax.lax.dot_general(xb, we_ref[...], (((1,), (0,)), ((), ())),
                                      preferred_element_type=jnp.float32)
    p_ref[0, 0] = xb + jax.lax.dot_general(
        xb, wo1_ref[...], (((1,), (0,)), ((), ())),
        preferred_element_type=jnp.float32) + bo_ref[...][None, :]


def _proj(x, W_edge, Wo1, b_offset):
    grid = (B, V)
    return pl.pallas_call(
        _proj_body,
        grid=grid,
        in_specs=[
            pl.BlockSpec((1, 1, N, D), lambda b, v: (b, v, 0, 0)),
            pl.BlockSpec((D, D), lambda b, v: (0, 0)),
            pl.BlockSpec((D, D), lambda b, v: (0, 0)),
            pl.BlockSpec((D,), lambda b, v: (0,)),
        ],
        out_specs=[
            pl.BlockSpec((1, 1, N, D), lambda b, v: (b, v, 0, 0)),
            pl.BlockSpec((1, 1, N, D), lambda b, v: (b, v, 0, 0)),
        ],
        out_shape=[
            jax.ShapeDtypeStruct((B, V, N, D), jnp.float32),
            jax.ShapeDtypeStruct((B, V, N, D), jnp.float32),
        ],
    )(x, W_edge, Wo1, b_offset)


QT = 49
NCH = NP // 16
GQ = 4


def _sc_topk_gather_body(scores_hbm, u_hbm, ug_hbm, b0, b1, b2, idx_v,
                         rows_v, rows_tail, sem):
    nc = 2
    wid = lax.axis_index("s") * nc + lax.axis_index("c")
    q0 = wid * QT
    b = q0 // M
    v = (q0 % M) // N
    n0 = q0 % N
    bufs = (b0, b1, b2)
    offs = []
    for j in range(NV):
        row0 = ((b * V + v) * NV + j) * N + n0
        al = (row0 // 8) * 8
        offs.append(row0 - al)
        pltpu.sync_copy(scores_hbm.at[pl.ds(al, QT + 7)], bufs[j])

    lane = lax.iota(jnp.int32, 16)
    inf16 = jnp.full((16,), jnp.inf, jnp.float32)
    zero16 = jnp.zeros((16,), jnp.int32)
    sentinel = jnp.where(lane < K, 0, jnp.int32(2**30))

    def per_query(qq, _):
        def per_chunk(c, st):
            out = []
            for j in range(NV):
                bv, bi = st[2 * j], st[2 * j + 1]
                vals = bufs[j][qq + offs[j], pl.ds(c * 16, 16)]
                idxs = c * 16 + lane
                sv, si = plsc.sort_key_val(vals, idxs)
                rv = lax.rev(sv, (0,))
                ri = lax.rev(si, (0,))
                keep = bv <= rv
                mv = jnp.where(keep, bv, rv)
                mi = jnp.where(keep, bi, ri)
                nbv, nbi = plsc.sort_key_val(mv, mi)
                out.extend([nbv, nbi])
            return tuple(out)

        st = (inf16, zero16) * NV
        st = lax.fori_loop(0, NCH, per_chunk, st)
        for j in range(NV):
            v1 = j + jnp.where(j >= v, 1, 0)
            base = b * M + v1 * N
            keys = st[2 * j + 1] + sentinel
            gidx, _ = plsc.sort_key_val(keys, keys)
            plsc.store_compressed(idx_v.at[pl.ds(qq * (NV * K) + j * K, 16)],
                                  gidx + base, mask=lane < K)
        return 0

    lax.fori_loop(0, QT, per_query, 0)

    out0 = q0 * NV * K
    for ch in range(QT // GQ):
        cp = pltpu.async_copy(
            u_hbm.at[idx_v.at[pl.ds(ch * GQ * NV * K, GQ * NV * K)]],
            rows_v, sem)
        cp.wait()
        pltpu.sync_copy(rows_v, ug_hbm.at[pl.ds(out0 + ch * GQ * NV * K,
                                                GQ * NV * K)])
    tail0 = (QT // GQ) * GQ * NV * K
    cp = pltpu.async_copy(u_hbm.at[idx_v.at[pl.ds(tail0, NV * K)]],
                          rows_tail, sem)
    cp.wait()
    pltpu.sync_copy(rows_tail, ug_hbm.at[pl.ds(out0 + tail0, NV * K)])


def _topk_gather_sc(scores, U):
    scores2 = scores.reshape(B * V * NV * N, NP)
    U2 = U.reshape(B * M, D)
    mesh = plsc.VectorSubcoreMesh(core_axis_name="c", subcore_axis_name="s")
    fn = pl.kernel(
        _sc_topk_gather_body,
        out_type=jax.ShapeDtypeStruct((B * M * NV * K, D), jnp.float32),
        mesh=mesh,
        scratch_types=[
            pltpu.VMEM((QT + 7, NP), jnp.float32),
            pltpu.VMEM((QT + 7, NP), jnp.float32),
            pltpu.VMEM((QT + 7, NP), jnp.float32),
            pltpu.VMEM((QT * NV * K + 8,), jnp.int32),
            pltpu.VMEM((GQ * NV * K, D), jnp.float32),
            pltpu.VMEM((NV * K, D), jnp.float32),
            pltpu.SemaphoreType.DMA,
        ],
        compiler_params=pltpu.CompilerParams(needs_layout_passes=False),
    )
    return fn(scores2, U2)


QC = 49


def _fusion_body(ug_ref, u_ref, p_ref, wf_ref, wo2_ref, be_ref, out_ref):
    ug = ug_ref[0]
    uq = u_ref[0]
    pre = (ug.reshape(QC, NV * K, D) - uq[:, None, :] + be_ref[...][None, None, :])
    edge = 0.5 * pre * (1.0 + lax.erf(pre * (2.0 ** -0.5)))
    logits = jax.lax.dot_general(
        edge.reshape(QC * NV * K, D), wf_ref[...], (((1,), (0,)), ((), ())),
        preferred_element_type=jnp.float32).reshape(QC, NV * K, D)
    mx = jnp.max(logits, axis=1, keepdims=True)
    e = jnp.exp(logits - mx)
    edge_sum = jnp.sum(edge * e, axis=1) / jnp.sum(e, axis=1)
    out_ref[0] = p_ref[0] + jax.lax.dot_general(
        edge_sum, wo2_ref[...], (((1,), (0,)), ((), ())),
        preferred_element_type=jnp.float32)


def _fusion(Ug, U, P, W_fusion, Wo2, b_edge):
    R = B * M
    grid = (R // QC,)
    return pl.pallas_call(
        _fusion_body,
        grid=grid,
        in_specs=[
            pl.BlockSpec((1, QC * NV * K, D), lambda i: (i, 0, 0)),
            pl.BlockSpec((1, QC, D), lambda i: (i, 0, 0)),
            pl.BlockSpec((1, QC, D), lambda i: (i, 0, 0)),
            pl.BlockSpec((D, D), lambda i: (0, 0)),
            pl.BlockSpec((D, D), lambda i: (0, 0)),
            pl.BlockSpec((D,), lambda i: (0,)),
        ],
        out_specs=pl.BlockSpec((1, QC, D), lambda i: (i, 0, 0)),
        out_shape=jax.ShapeDtypeStruct((R // QC, QC, D), jnp.float32),
    )(Ug.reshape(R // QC, QC * NV * K, D),
      U.reshape(R // QC, QC, D),
      P.reshape(R // QC, QC, D),
      W_fusion, Wo2, b_edge)


def kernel(x, W_edge, b_edge, W_fusion, W_offset, b_offset):
    scores = _scores(x)
    U, P = _proj(x, W_edge, W_offset[:D], b_offset)
    Ug = _topk_gather_sc(scores, U)
    out = _fusion(Ug.reshape(B * M, NV * K, D),
                  U.reshape(B * M, D),
                  P.reshape(B * M, D),
                  W_fusion, W_offset[D:], b_edge)
    return out.reshape(B, V, N, D)
